# baseline (speedup 1.0000x reference)
import functools

import jax
import jax.numpy as jnp
from jax import lax
from jax.experimental import pallas as pl
from jax.experimental.pallas import tpu as pltpu
from jax.experimental.pallas import tpu_sc as plsc

_ROWS = 1000000
_DIM = 64
_NWORKERS = 32
_CHUNK = 504
_NCHUNKS = 62
_SPAN = _CHUNK * _NCHUNKS
_TAIL = _ROWS - _SPAN * _NWORKERS
_NBUF = 2


def _sc_copy(w_hbm, out_hbm, buf, in_sems, out_sems):
    wid = lax.axis_index("s") * 2 + lax.axis_index("c")
    base = wid * _SPAN

    def in_copy(c, b):
        return pltpu.make_async_copy(
            w_hbm.at[pl.ds(base + c * _CHUNK, _CHUNK), :],
            buf.at[b],
            in_sems.at[b],
        )

    def out_copy(c, b):
        return pltpu.make_async_copy(
            buf.at[b],
            out_hbm.at[pl.ds(base + c * _CHUNK, _CHUNK), :],
            out_sems.at[b],
        )

    in_copy(0, 0).start()
    for c in range(_NCHUNKS):
        b = c % _NBUF
        in_copy(c, b).wait()
        out_copy(c, b).start()
        if c >= 1:
            out_copy(c - 1, 1 - b).wait()
        if c + 1 < _NCHUNKS:
            in_copy(c + 1, 1 - b).start()
    out_copy(_NCHUNKS - 1, (_NCHUNKS - 1) % _NBUF).wait()

    @pl.when(wid == _NWORKERS - 1)
    def _():
        tail_base = _SPAN * _NWORKERS
        tin = pltpu.make_async_copy(
            w_hbm.at[pl.ds(tail_base, _TAIL), :],
            buf.at[0, pl.ds(0, _TAIL), :],
            in_sems.at[0],
        )
        tin.start()
        tin.wait()
        tout = pltpu.make_async_copy(
            buf.at[0, pl.ds(0, _TAIL), :],
            out_hbm.at[pl.ds(tail_base, _TAIL), :],
            out_sems.at[0],
        )
        tout.start()
        tout.wait()


def kernel(weight):
    mesh = plsc.VectorSubcoreMesh(core_axis_name="c", subcore_axis_name="s")
    run = functools.partial(
        pl.kernel,
        mesh=mesh,
        out_type=jax.ShapeDtypeStruct((_ROWS, _DIM), jnp.float32),
        scratch_types=[
            pltpu.VMEM((_NBUF, _CHUNK, _DIM), jnp.float32),
            pltpu.SemaphoreType.DMA((_NBUF,)),
            pltpu.SemaphoreType.DMA((_NBUF,)),
        ],
    )(_sc_copy)
    return run(weight)
